# baseline (speedup 1.0000x reference)
import jax
import jax.numpy as jnp
from jax.experimental import pallas as pl

_NQ = 4
_NE = 1024
_ED = 32
_B = 16384
_BR = 256


def _rvq_block(x_ref, cb_ref, xq_ref, idx_ref, loss_ref):
    step = pl.program_id(0)

    @pl.when(step == 0)
    def _():
        loss_ref[...] = jnp.zeros_like(loss_ref)

    r = x_ref[...]
    xq_acc = jnp.zeros_like(r)
    loss_acc = jnp.zeros((), jnp.float32)
    lane = jax.lax.broadcasted_iota(jnp.int32, (_BR, _NE), 1)
    ones_row = jnp.ones((1, _ED), jnp.float32)
    for i in range(_NQ):
        cb = cb_ref[i]
        csum = jax.lax.dot_general(ones_row, cb * cb, (((1,), (1,)), ((), ())),
                                   precision=jax.lax.Precision.HIGHEST,
                                   preferred_element_type=jnp.float32)
        m = jax.lax.dot_general(r, cb, (((1,), (1,)), ((), ())),
                                preferred_element_type=jnp.float32)
        rsum = jnp.sum(r * r, axis=1, keepdims=True)
        d = (rsum + csum) - 2.0 * m
        dmin = jnp.min(d, axis=1, keepdims=True)
        cand = jnp.where(d <= dmin, lane, _NE)
        idxc = jnp.min(cand, axis=1, keepdims=True)
        oh = (lane == idxc).astype(jnp.float32)
        xq = jax.lax.dot_general(oh, cb, (((1,), (0,)), ((), ())),
                                 precision=jax.lax.Precision.HIGHEST,
                                 preferred_element_type=jnp.float32)
        t = xq - r
        xq_st = r + t
        r = r - xq_st
        xq_acc = xq_acc + xq_st
        loss_acc = loss_acc + jnp.sum(t * t)
        idx_ref[:, i:i + 1] = idxc
    xq_ref[...] = xq_acc
    loss_ref[...] = loss_ref[...] + loss_acc


@jax.jit
def kernel(x, labels, codebooks):
    del labels
    xq, idxs, loss = pl.pallas_call(
        _rvq_block,
        grid=(_B // _BR,),
        in_specs=[
            pl.BlockSpec((_BR, _ED), lambda i: (i, 0)),
            pl.BlockSpec((_NQ, _NE, _ED), lambda i: (0, 0, 0)),
        ],
        out_specs=[
            pl.BlockSpec((_BR, _ED), lambda i: (i, 0)),
            pl.BlockSpec((_BR, _NQ), lambda i: (i, 0)),
            pl.BlockSpec((1, 128), lambda i: (0, 0)),
        ],
        out_shape=[
            jax.ShapeDtypeStruct((_B, _ED), jnp.float32),
            jax.ShapeDtypeStruct((_B, _NQ), jnp.int32),
            jax.ShapeDtypeStruct((1, 128), jnp.float32),
        ],
    )(x, codebooks)
    mean_losses = loss[0, 0] * (2.0 / (_NQ * _B * _ED))
    return xq, mean_losses, idxs


# speedup vs baseline: 2.6043x; 2.6043x over previous
import jax
import jax.numpy as jnp
from jax.experimental import pallas as pl
from jax.experimental.pallas import tpu as pltpu

_NQ = 4
_NE = 1024
_ED = 32
_B = 16384
_BR = 512


def _rvq_block(x_ref, cb_ref, xq_ref, idx_ref, loss_ref, csum_ref, cbs_ref):
    step = pl.program_id(0)

    @pl.when(step == 0)
    def _():
        loss_ref[...] = jnp.zeros_like(loss_ref)
        ones_row = jnp.ones((1, _ED), jnp.float32)
        for i in range(_NQ):
            cb = cb_ref[i]
            csum_ref[i:i + 1, :] = jax.lax.dot_general(
                ones_row, cb * cb, (((1,), (1,)), ((), ())),
                precision=jax.lax.Precision.HIGHEST,
                preferred_element_type=jnp.float32)
            hi = cb.astype(jnp.bfloat16)
            rem = cb - hi.astype(jnp.float32)
            lo = rem.astype(jnp.bfloat16)
            llo = (rem - lo.astype(jnp.float32)).astype(jnp.bfloat16)
            cbs_ref[i] = jnp.concatenate([hi, lo, llo], axis=1)

    r = x_ref[...]
    xq_acc = jnp.zeros_like(r)
    loss_acc = jnp.zeros((), jnp.float32)
    lane = jax.lax.broadcasted_iota(jnp.int32, (_BR, _NE), 1)
    for i in range(_NQ):
        cb = cb_ref[i]
        csum = csum_ref[i:i + 1, :]
        m = jax.lax.dot_general(r, cb, (((1,), (1,)), ((), ())),
                                preferred_element_type=jnp.float32)
        rsum = jnp.sum(r * r, axis=1, keepdims=True)
        d = (rsum + csum) - 2.0 * m
        dmin = jnp.min(d, axis=1, keepdims=True)
        cand = jnp.where(d <= dmin, lane, _NE)
        idxc = jnp.min(cand, axis=1, keepdims=True)
        oh = (lane == idxc).astype(jnp.bfloat16)
        xq3 = jax.lax.dot_general(oh, cbs_ref[i], (((1,), (0,)), ((), ())),
                                  preferred_element_type=jnp.float32)
        xq = (xq3[:, 0:_ED] + xq3[:, _ED:2 * _ED]) + xq3[:, 2 * _ED:3 * _ED]
        t = xq - r
        xq_st = r + t
        r = r - xq_st
        xq_acc = xq_acc + xq_st
        loss_acc = loss_acc + jnp.sum(t * t)
        idx_ref[:, i:i + 1] = idxc
    xq_ref[...] = xq_acc
    loss_ref[...] = loss_ref[...] + loss_acc


@jax.jit
def kernel(x, labels, codebooks):
    del labels
    xq, idxs, loss = pl.pallas_call(
        _rvq_block,
        grid=(_B // _BR,),
        in_specs=[
            pl.BlockSpec((_BR, _ED), lambda i: (i, 0)),
            pl.BlockSpec((_NQ, _NE, _ED), lambda i: (0, 0, 0)),
        ],
        out_specs=[
            pl.BlockSpec((_BR, _ED), lambda i: (i, 0)),
            pl.BlockSpec((_BR, _NQ), lambda i: (i, 0)),
            pl.BlockSpec((1, 128), lambda i: (0, 0)),
        ],
        out_shape=[
            jax.ShapeDtypeStruct((_B, _ED), jnp.float32),
            jax.ShapeDtypeStruct((_B, _NQ), jnp.int32),
            jax.ShapeDtypeStruct((1, 128), jnp.float32),
        ],
        scratch_shapes=[
            pltpu.VMEM((_NQ, _NE), jnp.float32),
            pltpu.VMEM((_NQ, _NE, 3 * _ED), jnp.bfloat16),
        ],
    )(x, codebooks)
    mean_losses = loss[0, 0] * (2.0 / (_NQ * _B * _ED))
    return xq, mean_losses, idxs


# speedup vs baseline: 2.9045x; 1.1153x over previous
import jax
import jax.numpy as jnp
from jax.experimental import pallas as pl
from jax.experimental.pallas import tpu as pltpu

_NQ = 4
_NE = 1024
_ED = 32
_B = 16384
_BR = 1024


def _rvq_block(x_ref, cb_ref, xq_ref, idx_ref, loss_ref, csum_ref, cbs_ref):
    step = pl.program_id(0)

    @pl.when(step == 0)
    def _():
        loss_ref[...] = jnp.zeros_like(loss_ref)
        ones_row = jnp.ones((1, _ED), jnp.float32)
        for i in range(_NQ):
            cb = cb_ref[i]
            csum_ref[i:i + 1, :] = jax.lax.dot_general(
                ones_row, cb * cb, (((1,), (1,)), ((), ())),
                precision=jax.lax.Precision.HIGHEST,
                preferred_element_type=jnp.float32)
            hi = cb.astype(jnp.bfloat16)
            rem = cb - hi.astype(jnp.float32)
            lo = rem.astype(jnp.bfloat16)
            llo = (rem - lo.astype(jnp.float32)).astype(jnp.bfloat16)
            cbs_ref[i] = jnp.concatenate([hi, lo, llo], axis=1)

    r = x_ref[...]
    xq_acc = jnp.zeros_like(r)
    loss_acc = jnp.zeros((), jnp.float32)
    lane = jax.lax.broadcasted_iota(jnp.int32, (_BR, _NE), 1)
    for i in range(_NQ):
        cb = cb_ref[i]
        csum = csum_ref[i:i + 1, :]
        m2 = jax.lax.dot_general(r + r, cb, (((1,), (1,)), ((), ())),
                                 preferred_element_type=jnp.float32)
        rsum = jnp.sum(r * r, axis=1, keepdims=True)
        d = (rsum + csum) - m2
        dmin = jnp.min(d, axis=1, keepdims=True)
        cand = jnp.where(d <= dmin, lane, _NE)
        idxc = jnp.min(cand, axis=1, keepdims=True)
        oh = (lane == idxc).astype(jnp.bfloat16)
        xq3 = jax.lax.dot_general(oh, cbs_ref[i], (((1,), (0,)), ((), ())),
                                  preferred_element_type=jnp.float32)
        xq = (xq3[:, 0:_ED] + xq3[:, _ED:2 * _ED]) + xq3[:, 2 * _ED:3 * _ED]
        t = xq - r
        xq_st = r + t
        r = r - xq_st
        xq_acc = xq_acc + xq_st
        loss_acc = loss_acc + jnp.sum(t * t)
        idx_ref[:, i:i + 1] = idxc
    xq_ref[...] = xq_acc
    loss_ref[...] = loss_ref[...] + loss_acc


@jax.jit
def kernel(x, labels, codebooks):
    del labels
    xq, idxs, loss = pl.pallas_call(
        _rvq_block,
        grid=(_B // _BR,),
        in_specs=[
            pl.BlockSpec((_BR, _ED), lambda i: (i, 0)),
            pl.BlockSpec((_NQ, _NE, _ED), lambda i: (0, 0, 0)),
        ],
        out_specs=[
            pl.BlockSpec((_BR, _ED), lambda i: (i, 0)),
            pl.BlockSpec((_BR, _NQ), lambda i: (i, 0)),
            pl.BlockSpec((1, 128), lambda i: (0, 0)),
        ],
        out_shape=[
            jax.ShapeDtypeStruct((_B, _ED), jnp.float32),
            jax.ShapeDtypeStruct((_B, _NQ), jnp.int32),
            jax.ShapeDtypeStruct((1, 128), jnp.float32),
        ],
        scratch_shapes=[
            pltpu.VMEM((_NQ, _NE), jnp.float32),
            pltpu.VMEM((_NQ, _NE, 3 * _ED), jnp.bfloat16),
        ],
    )(x, codebooks)
    mean_losses = loss[0, 0] * (2.0 / (_NQ * _B * _ED))
    return xq, mean_losses, idxs


# speedup vs baseline: 2.9760x; 1.0246x over previous
import jax
import jax.numpy as jnp
from jax.experimental import pallas as pl
from jax.experimental.pallas import tpu as pltpu

_NQ = 4
_NE = 1024
_ED = 32
_B = 16384
_BR = 1024
_NA = 128


def _rvq_block(x_ref, cb_ref, xq_ref, idx_ref, loss_ref,
               csum_ref, cbs_ref, xqs_ref, idxs_ref):
    step = pl.program_id(0)

    @pl.when(step == 0)
    def _():
        loss_ref[...] = jnp.zeros_like(loss_ref)
        ones_row = jnp.ones((1, _ED), jnp.float32)
        lane_col = jax.lax.broadcasted_iota(jnp.int32, (_NE, 1), 0)
        ones_col = jnp.ones((_NE, 1), jnp.bfloat16)
        lane_hi = (lane_col >> 5).astype(jnp.bfloat16)
        lane_lo = (lane_col & 31).astype(jnp.bfloat16)
        pad = jnp.zeros((_NE, _NA - 3 * _ED - 3), jnp.bfloat16)
        for i in range(_NQ):
            cb = cb_ref[i]
            csum_ref[i:i + 1, :] = jax.lax.dot_general(
                ones_row, cb * cb, (((1,), (1,)), ((), ())),
                precision=jax.lax.Precision.HIGHEST,
                preferred_element_type=jnp.float32)
            hi = cb.astype(jnp.bfloat16)
            rem = cb - hi.astype(jnp.float32)
            lo = rem.astype(jnp.bfloat16)
            llo = (rem - lo.astype(jnp.float32)).astype(jnp.bfloat16)
            cbs_ref[i] = jnp.concatenate(
                [hi, lo, llo, ones_col, lane_hi, lane_lo, pad], axis=1)

    r = x_ref[...]
    xq_acc = jnp.zeros_like(r)
    loss_acc = jnp.zeros((), jnp.float32)
    lane = jax.lax.broadcasted_iota(jnp.int32, (_BR, _NE), 1)
    for i in range(_NQ):
        cb = cb_ref[i]
        csum = csum_ref[i:i + 1, :]
        m2 = jax.lax.dot_general(r + r, cb, (((1,), (1,)), ((), ())),
                                 preferred_element_type=jnp.float32)
        rsum = jnp.sum(r * r, axis=1, keepdims=True)
        d = (rsum + csum) - m2
        dmin = jnp.min(d, axis=1, keepdims=True)
        mask = (d <= dmin).astype(jnp.bfloat16)
        g = jax.lax.dot_general(mask, cbs_ref[i], (((1,), (0,)), ((), ())),
                                preferred_element_type=jnp.float32)
        xqs_ref[...] = (g[:, 0:_ED] + g[:, _ED:2 * _ED]) + g[:, 2 * _ED:3 * _ED]
        cnt = g[:, 3 * _ED:3 * _ED + 1]
        idxs_ref[...] = (g[:, 3 * _ED + 1:3 * _ED + 2] * 32.0
                         + g[:, 3 * _ED + 2:3 * _ED + 3]).astype(jnp.int32)
        has_tie = jnp.max(cnt) > 1.0

        @pl.when(has_tie)
        def _():
            cand = jnp.where(d <= dmin, lane, _NE)
            idxc = jnp.min(cand, axis=1, keepdims=True)
            oh = (lane == idxc).astype(jnp.bfloat16)
            g2 = jax.lax.dot_general(oh, cbs_ref[i], (((1,), (0,)), ((), ())),
                                     preferred_element_type=jnp.float32)
            xqs_ref[...] = ((g2[:, 0:_ED] + g2[:, _ED:2 * _ED])
                            + g2[:, 2 * _ED:3 * _ED])
            idxs_ref[...] = idxc

        xq = xqs_ref[...]
        t = xq - r
        xq_st = r + t
        r = r - xq_st
        xq_acc = xq_acc + xq_st
        loss_acc = loss_acc + jnp.sum(t * t)
        idx_ref[:, i:i + 1] = idxs_ref[...]
    xq_ref[...] = xq_acc
    loss_ref[...] = loss_ref[...] + loss_acc


@jax.jit
def kernel(x, labels, codebooks):
    del labels
    xq, idxs, loss = pl.pallas_call(
        _rvq_block,
        grid=(_B // _BR,),
        in_specs=[
            pl.BlockSpec((_BR, _ED), lambda i: (i, 0)),
            pl.BlockSpec((_NQ, _NE, _ED), lambda i: (0, 0, 0)),
        ],
        out_specs=[
            pl.BlockSpec((_BR, _ED), lambda i: (i, 0)),
            pl.BlockSpec((_BR, _NQ), lambda i: (i, 0)),
            pl.BlockSpec((1, 128), lambda i: (0, 0)),
        ],
        out_shape=[
            jax.ShapeDtypeStruct((_B, _ED), jnp.float32),
            jax.ShapeDtypeStruct((_B, _NQ), jnp.int32),
            jax.ShapeDtypeStruct((1, 128), jnp.float32),
        ],
        scratch_shapes=[
            pltpu.VMEM((_NQ, _NE), jnp.float32),
            pltpu.VMEM((_NQ, _NE, _NA), jnp.bfloat16),
            pltpu.VMEM((_BR, _ED), jnp.float32),
            pltpu.VMEM((_BR, 1), jnp.int32),
        ],
    )(x, codebooks)
    mean_losses = loss[0, 0] * (2.0 / (_NQ * _B * _ED))
    return xq, mean_losses, idxs


# speedup vs baseline: 3.1900x; 1.0719x over previous
import jax
import jax.numpy as jnp
from jax.experimental import pallas as pl
from jax.experimental.pallas import tpu as pltpu

_NQ = 4
_NE = 1024
_ED = 32
_B = 16384
_BR = 2048
_NA = 128


def _rvq_block(x_ref, cb_ref, xq_ref, idx_ref, loss_ref,
               csum_ref, cbs_ref, xqs_ref, idxs_ref):
    step = pl.program_id(0)

    @pl.when(step == 0)
    def _():
        loss_ref[...] = jnp.zeros_like(loss_ref)
        ones_row = jnp.ones((1, _ED), jnp.float32)
        lane_col = jax.lax.broadcasted_iota(jnp.int32, (_NE, 1), 0)
        ones_col = jnp.ones((_NE, 1), jnp.bfloat16)
        lane_hi = (lane_col >> 5).astype(jnp.bfloat16)
        lane_lo = (lane_col & 31).astype(jnp.bfloat16)
        pad = jnp.zeros((_NE, _NA - 3 * _ED - 3), jnp.bfloat16)
        for i in range(_NQ):
            cb = cb_ref[i]
            csum_ref[i:i + 1, :] = jax.lax.dot_general(
                ones_row, cb * cb, (((1,), (1,)), ((), ())),
                precision=jax.lax.Precision.HIGHEST,
                preferred_element_type=jnp.float32)
            hi = cb.astype(jnp.bfloat16)
            rem = cb - hi.astype(jnp.float32)
            lo = rem.astype(jnp.bfloat16)
            llo = (rem - lo.astype(jnp.float32)).astype(jnp.bfloat16)
            cbs_ref[i] = jnp.concatenate(
                [hi, lo, llo, ones_col, lane_hi, lane_lo, pad], axis=1)

    r = x_ref[...]
    xq_acc = jnp.zeros_like(r)
    loss_acc = jnp.zeros((), jnp.float32)
    lane = jax.lax.broadcasted_iota(jnp.int32, (_BR, _NE), 1)
    for i in range(_NQ):
        cb = cb_ref[i]
        csum = csum_ref[i:i + 1, :]
        m2 = jax.lax.dot_general(r + r, cb, (((1,), (1,)), ((), ())),
                                 preferred_element_type=jnp.float32)
        rsum = jnp.sum(r * r, axis=1, keepdims=True)
        d = (rsum + csum) - m2
        dmin = jnp.min(d, axis=1, keepdims=True)
        mask = (d <= dmin).astype(jnp.bfloat16)
        g = jax.lax.dot_general(mask, cbs_ref[i], (((1,), (0,)), ((), ())),
                                preferred_element_type=jnp.float32)
        xqs_ref[...] = (g[:, 0:_ED] + g[:, _ED:2 * _ED]) + g[:, 2 * _ED:3 * _ED]
        cnt = g[:, 3 * _ED:3 * _ED + 1]
        idxs_ref[...] = (g[:, 3 * _ED + 1:3 * _ED + 2] * 32.0
                         + g[:, 3 * _ED + 2:3 * _ED + 3]).astype(jnp.int32)
        has_tie = jnp.max(cnt) > 1.0

        @pl.when(has_tie)
        def _():
            cand = jnp.where(d <= dmin, lane, _NE)
            idxc = jnp.min(cand, axis=1, keepdims=True)
            oh = (lane == idxc).astype(jnp.bfloat16)
            g2 = jax.lax.dot_general(oh, cbs_ref[i], (((1,), (0,)), ((), ())),
                                     preferred_element_type=jnp.float32)
            xqs_ref[...] = ((g2[:, 0:_ED] + g2[:, _ED:2 * _ED])
                            + g2[:, 2 * _ED:3 * _ED])
            idxs_ref[...] = idxc

        xq = xqs_ref[...]
        t = xq - r
        xq_st = r + t
        r = r - xq_st
        xq_acc = xq_acc + xq_st
        loss_acc = loss_acc + jnp.sum(t * t)
        idx_ref[:, i:i + 1] = idxs_ref[...]
    xq_ref[...] = xq_acc
    loss_ref[...] = loss_ref[...] + loss_acc


@jax.jit
def kernel(x, labels, codebooks):
    del labels
    xq, idxs, loss = pl.pallas_call(
        _rvq_block,
        grid=(_B // _BR,),
        in_specs=[
            pl.BlockSpec((_BR, _ED), lambda i: (i, 0)),
            pl.BlockSpec((_NQ, _NE, _ED), lambda i: (0, 0, 0)),
        ],
        out_specs=[
            pl.BlockSpec((_BR, _ED), lambda i: (i, 0)),
            pl.BlockSpec((_BR, _NQ), lambda i: (i, 0)),
            pl.BlockSpec((1, 128), lambda i: (0, 0)),
        ],
        out_shape=[
            jax.ShapeDtypeStruct((_B, _ED), jnp.float32),
            jax.ShapeDtypeStruct((_B, _NQ), jnp.int32),
            jax.ShapeDtypeStruct((1, 128), jnp.float32),
        ],
        scratch_shapes=[
            pltpu.VMEM((_NQ, _NE), jnp.float32),
            pltpu.VMEM((_NQ, _NE, _NA), jnp.bfloat16),
            pltpu.VMEM((_BR, _ED), jnp.float32),
            pltpu.VMEM((_BR, 1), jnp.int32),
        ],
    )(x, codebooks)
    mean_losses = loss[0, 0] * (2.0 / (_NQ * _B * _ED))
    return xq, mean_losses, idxs
